# baseline (speedup 1.0000x reference)
import functools

import jax
import jax.numpy as jnp
from jax import lax
from jax.experimental import pallas as pl
from jax.experimental.pallas import tpu as pltpu
from jax.experimental.pallas import tpu_sc as plsc

N = 10000
E = 320000
D = 128
NRBF = 20
T = 3
CUTOFF = 5.0
MAX_Z = 100

NC = 2
NS = 16
NW = NC * NS
EPW = E // NW
C = 40
NCH = EPW // C
NBUF = 2
NP = 10240
SR = NP // NS

BE = 4000
BN = 1000

_F32 = jnp.float32


def _ssp(x):
    return jax.nn.softplus(x) - jnp.log(2.0).astype(_F32)


S = 4096
TPAD = 8
DELTA = CUTOFF / S


def _tables_body(wf1_ref, bf1_ref, wf2_ref, bf2_ref, o0, o1, o2):
    k = lax.broadcasted_iota(jnp.int32, (S, 1), 0).astype(_F32)
    d = (k + 0.5) * DELTA
    step = CUTOFF / (NRBF - 1)
    offsets = lax.broadcasted_iota(jnp.int32, (1, NRBF), 1).astype(_F32) * step
    coeff = -0.5 / step**2
    f = jnp.exp(coeff * (d - offsets) ** 2)
    rcut = 0.5 * (jnp.cos(d * (jnp.pi / CUTOFF)) + 1.0)
    rcut = rcut * (d < CUTOFF).astype(_F32)
    wf1 = wf1_ref[...]
    bf1 = bf1_ref[...]
    wf2 = wf2_ref[...]
    bf2 = bf2_ref[...]
    for t, o in enumerate((o0, o1, o2)):
        y = _ssp(jnp.dot(f, wf1[t], preferred_element_type=_F32) + bf1[t][None, :])
        w = jnp.dot(y, wf2[t], preferred_element_type=_F32) + bf2[t][None, :]
        o[...] = w * rcut


def _tables(Wf1, bf1, Wf2, bf2):
    sb = pl.BlockSpec((S, D), lambda: (0, 0))
    return pl.pallas_call(
        _tables_body,
        in_specs=[
            pl.BlockSpec((T, NRBF, D), lambda: (0, 0, 0)),
            pl.BlockSpec((T, D), lambda: (0, 0)),
            pl.BlockSpec((T, D, D), lambda: (0, 0, 0)),
            pl.BlockSpec((T, D), lambda: (0, 0)),
        ],
        out_specs=[sb, sb, sb],
        out_shape=[jax.ShapeDtypeStruct((S, D), _F32)] * T,
    )(Wf1, bf1, Wf2, bf2)


def _bins_body(rij_ref, o_ref):
    r = rij_ref[...]
    d = jnp.sqrt(jnp.sum(r * r, axis=1, keepdims=True) + 1e-12)
    b = jnp.minimum((d * (1.0 / DELTA)).astype(jnp.int32), S)
    o_ref[...] = b


def _bins(Rij):
    return pl.pallas_call(
        _bins_body,
        grid=(E // BE,),
        in_specs=[pl.BlockSpec((BE, 3), lambda i: (i, 0))],
        out_specs=pl.BlockSpec((BE, 1), lambda i: (i, 0)),
        out_shape=jax.ShapeDtypeStruct((E, 1), jnp.int32),
    )(Rij)


def _embed_body(z_ref, emb_ref, w_ref, x_out, h_out):
    z = z_ref[...]
    classes = lax.broadcasted_iota(jnp.int32, (BN, MAX_Z), 1)
    onehot = (z == classes).astype(_F32)
    x = jnp.dot(onehot, emb_ref[...], preferred_element_type=_F32)
    x_out[...] = x
    h_out[...] = jnp.dot(x, w_ref[...], preferred_element_type=_F32)


def _embed_init(Z, emb, w_in2f0):
    grid = (N // BN,)
    nb = pl.BlockSpec((BN, D), lambda i: (i, 0))
    return pl.pallas_call(
        _embed_body,
        grid=grid,
        in_specs=[
            pl.BlockSpec((BN, 1), lambda i: (i, 0)),
            pl.BlockSpec((MAX_Z, D), lambda i: (0, 0)),
            pl.BlockSpec((D, D), lambda i: (0, 0)),
        ],
        out_specs=[nb, nb],
        out_shape=[jax.ShapeDtypeStruct((N, D), _F32)] * 2,
    )(Z.reshape(N, 1).astype(jnp.int32), emb, w_in2f0)


def _sc_edge_body(h_hbm, wtab_hbm, bins_hbm, idxi_hbm, idxj_hbm, zrows_hbm,
                  out_hbm,
                  idxj_all, bins_all,
                  idxi0, idxi1, rows0, rows1, w0, w1, agg_sh,
                  gs0, gs1, ws0, ws1, is0, is1):
    cid = lax.axis_index("c")
    sid = lax.axis_index("s")
    wid = cid * NS + sid
    ebase = wid * EPW
    idxis = (idxi0, idxi1)
    rows = (rows0, rows1)
    ws = (w0, w1)
    gsems = (gs0, gs1)
    wsems = (ws0, ws1)
    isems = (is0, is1)

    pltpu.sync_copy(idxj_hbm.at[pl.ds(pl.multiple_of(ebase, 8), EPW)],
                    idxj_all)
    pltpu.sync_copy(bins_hbm.at[pl.ds(pl.multiple_of(ebase, 8), EPW)],
                    bins_all)
    pltpu.sync_copy(zrows_hbm, agg_sh.at[pl.ds(sid * SR, SR)])
    plsc.subcore_barrier()

    def fire(ck, b):
        off = pl.multiple_of(ebase + ck * C, 8)
        loc = pl.multiple_of(ck * C, 8)
        pltpu.async_copy(idxi_hbm.at[pl.ds(off, C)], idxis[b], isems[b])
        pltpu.async_copy(wtab_hbm.at[bins_all.at[pl.ds(loc, C)]], ws[b],
                         wsems[b])
        pltpu.async_copy(h_hbm.at[idxj_all.at[pl.ds(loc, C)]], rows[b],
                         gsems[b])

    def process(ck, b):
        off = pl.multiple_of(ebase + ck * C, 8)
        loc = pl.multiple_of(ck * C, 8)
        pltpu.make_async_copy(wtab_hbm.at[bins_all.at[pl.ds(loc, C)]], ws[b],
                              wsems[b]).wait()
        pltpu.make_async_copy(h_hbm.at[idxj_all.at[pl.ds(loc, C)]], rows[b],
                              gsems[b]).wait()

        @plsc.parallel_loop(0, C, 1, unroll=2)
        def _(i):
            for j in range(D // 16):
                sl = pl.ds(j * 16, 16)
                rows[b][i, sl] = rows[b][i, sl] * ws[b][i, sl]

        pltpu.make_async_copy(idxi_hbm.at[pl.ds(off, C)], idxis[b],
                              isems[b]).wait()
        pltpu.sync_copy(rows[b], agg_sh.at[idxis[b]], add=True)
        nxt = ck + NBUF

        @pl.when(nxt < NCH)
        def _():
            fire(nxt, b)

    for b in range(NBUF):
        fire(b, b)

    def outer(k, c):
        for b in range(NBUF):
            process(k * NBUF + b, b)
        return c

    lax.fori_loop(0, NCH // NBUF, outer, 0)
    plsc.subcore_barrier()

    pltpu.sync_copy(agg_sh.at[pl.ds(sid * SR, SR)],
                    out_hbm.at[cid, pl.ds(sid * SR, SR)])


def _sc_edge(h, wtab, bins, idx_i, idx_j, zrows):
    mesh = plsc.VectorSubcoreMesh(core_axis_name="c", subcore_axis_name="s")
    kern = pl.kernel(
        _sc_edge_body,
        out_type=jax.ShapeDtypeStruct((NC, NP, D), _F32),
        mesh=mesh,
        scratch_types=(
            [pltpu.VMEM((EPW,), jnp.int32),
             pltpu.VMEM((EPW,), jnp.int32)]
            + [pltpu.VMEM((C,), jnp.int32)] * NBUF
            + [pltpu.VMEM((C, D), _F32)] * (2 * NBUF)
            + [pltpu.VMEM_SHARED((NP, D), _F32)]
            + [pltpu.SemaphoreType.DMA] * (3 * NBUF)
        ),
    )
    return kern(h, wtab, bins, idx_i, idx_j, zrows)


def _mlp_body(agg_ref, x_ref, wo1_ref, bo1_ref, wo2_ref, bo2_ref, wnext_ref,
              x_out, h_out):
    a = agg_ref[0] + agg_ref[1]
    y = _ssp(jnp.dot(a, wo1_ref[...], preferred_element_type=_F32) + bo1_ref[...])
    v = jnp.dot(y, wo2_ref[...], preferred_element_type=_F32) + bo2_ref[...]
    xn = x_ref[...] + v
    x_out[...] = xn
    h_out[...] = jnp.dot(xn, wnext_ref[...], preferred_element_type=_F32)


def _mlp(agg, x, wo1, bo1, wo2, bo2, wnext):
    grid = (N // BN,)
    nb = pl.BlockSpec((BN, D), lambda i: (i, 0))
    full = pl.BlockSpec((D, D), lambda i: (0, 0))
    bias = pl.BlockSpec((1, D), lambda i: (0, 0))
    return pl.pallas_call(
        _mlp_body,
        grid=grid,
        in_specs=[
            pl.BlockSpec((NC, BN, D), lambda i: (0, i, 0)),
            nb, full, bias, full, bias, full,
        ],
        out_specs=[nb, nb],
        out_shape=[jax.ShapeDtypeStruct((N, D), _F32)] * 2,
    )(agg, x, wo1, bo1.reshape(1, D), wo2, bo2.reshape(1, D), wnext)


def kernel(Z, Rij, idx_i, idx_j, emb, W_in2f, Wf1, bf1, Wf2, bf2,
           Wo1, bo1, Wo2, bo2):
    idx_i = idx_i.astype(jnp.int32)
    idx_j = idx_j.astype(jnp.int32)
    tabs = _tables(Wf1, bf1, Wf2, bf2)
    zpad = jnp.zeros((TPAD, D), _F32)
    tabs = [jnp.concatenate([tab, zpad], axis=0) for tab in tabs]
    bins = _bins(Rij).reshape(E)
    x, h = _embed_init(Z, emb, W_in2f[0])
    zrows = jnp.zeros((SR, D), _F32)
    for t in range(T):
        agg = _sc_edge(h, tabs[t], bins, idx_i, idx_j, zrows)
        x, h = _mlp(agg, x, Wo1[t], bo1[t], Wo2[t], bo2[t],
                    W_in2f[(t + 1) % T])
    return x


# speedup vs baseline: 1.0461x; 1.0461x over previous
import functools

import jax
import jax.numpy as jnp
import numpy as np
from jax import lax
from jax.experimental import pallas as pl
from jax.experimental.pallas import tpu as pltpu
from jax.experimental.pallas import tpu_sc as plsc

N = 10000
E = 320000
D = 128
NRBF = 20
T = 3
CUTOFF = 5.0
MAX_Z = 100

NC = 2
NS = 16
NW = NC * NS
EPW = E // NW
C = 40
NCH = EPW // C
NBUF = 2
NP = 10240
SR = NP // NS

BE = 4000
BN = 1000

_F32 = jnp.float32

def _ssp(x):
    return jax.nn.softplus(x) - jnp.log(2.0).astype(_F32)


S = 4096
TPAD = 8
DELTA = CUTOFF / S


def _tables_body(wf1_ref, bf1_ref, wf2_ref, bf2_ref, o0, o1, o2):
    k = lax.broadcasted_iota(jnp.int32, (S, 1), 0).astype(_F32)
    d = (k + 0.5) * DELTA
    step = CUTOFF / (NRBF - 1)
    offsets = lax.broadcasted_iota(jnp.int32, (1, NRBF), 1).astype(_F32) * step
    coeff = -0.5 / step**2
    f = jnp.exp(coeff * (d - offsets) ** 2)
    rcut = 0.5 * (jnp.cos(d * (jnp.pi / CUTOFF)) + 1.0)
    rcut = rcut * (d < CUTOFF).astype(_F32)
    wf1 = wf1_ref[...]
    bf1 = bf1_ref[...]
    wf2 = wf2_ref[...]
    bf2 = bf2_ref[...]
    for t, o in enumerate((o0, o1, o2)):
        y = _ssp(jnp.dot(f, wf1[t], preferred_element_type=_F32) + bf1[t][None, :])
        w = jnp.dot(y, wf2[t], preferred_element_type=_F32) + bf2[t][None, :]
        o[...] = w * rcut


def _tables(Wf1, bf1, Wf2, bf2):
    sb = pl.BlockSpec((S, D), lambda: (0, 0))
    return pl.pallas_call(
        _tables_body,
        in_specs=[
            pl.BlockSpec((T, NRBF, D), lambda: (0, 0, 0)),
            pl.BlockSpec((T, D), lambda: (0, 0)),
            pl.BlockSpec((T, D, D), lambda: (0, 0, 0)),
            pl.BlockSpec((T, D), lambda: (0, 0)),
        ],
        out_specs=[sb, sb, sb],
        out_shape=[jax.ShapeDtypeStruct((S, D), _F32)] * T,
    )(Wf1, bf1, Wf2, bf2)


def _bins_body(rij_ref, o_ref):
    r = rij_ref[...]
    d = jnp.sqrt(jnp.sum(r * r, axis=1, keepdims=True) + 1e-12)
    b = jnp.minimum((d * (1.0 / DELTA)).astype(jnp.int32), S)
    o_ref[...] = b


def _bins(Rij):
    return pl.pallas_call(
        _bins_body,
        grid=(E // BE,),
        in_specs=[pl.BlockSpec((BE, 3), lambda i: (i, 0))],
        out_specs=pl.BlockSpec((BE, 1), lambda i: (i, 0)),
        out_shape=jax.ShapeDtypeStruct((E, 1), jnp.int32),
    )(Rij)


def _embed_body(z_ref, emb_ref, w_ref, x_out, h_out):
    z = z_ref[...]
    classes = lax.broadcasted_iota(jnp.int32, (BN, MAX_Z), 1)
    onehot = (z == classes).astype(_F32)
    x = jnp.dot(onehot, emb_ref[...], preferred_element_type=_F32)
    x_out[...] = x
    h_out[...] = jnp.dot(x, w_ref[...], preferred_element_type=_F32)


def _embed_init(Z, emb, w_in2f0):
    grid = (N // BN,)
    nb = pl.BlockSpec((BN, D), lambda i: (i, 0))
    return pl.pallas_call(
        _embed_body,
        grid=grid,
        in_specs=[
            pl.BlockSpec((BN, 1), lambda i: (i, 0)),
            pl.BlockSpec((MAX_Z, D), lambda i: (0, 0)),
            pl.BlockSpec((D, D), lambda i: (0, 0)),
        ],
        out_specs=[nb, nb],
        out_shape=[jax.ShapeDtypeStruct((N, D), _F32)] * 2,
    )(Z.reshape(N, 1).astype(jnp.int32), emb, w_in2f0)


def _sc_edge_body(h_hbm, wtab_hbm, bins_hbm, idxi_hbm, idxj_hbm, zrows_hbm,
                  out_hbm,
                  bins_all, hb0, hb1, wb0, wb1, pr0, pr1,
                  ij0, ij1, ij2, ij3, ij4, ii0, ii1, ii2, ii3, ii4,
                  agg_sh,
                  gs0, gs1, ws0, ws1, ss0, ss1, is0, is1, is2, is3, is4):
    cid = lax.axis_index("c")
    sid = lax.axis_index("s")
    wid = cid * NS + sid
    ebase = wid * EPW
    hbs = (hb0, hb1)
    wbs = (wb0, wb1)
    prods = (pr0, pr1)
    ijs = (ij0, ij1, ij2, ij3, ij4)
    iis = (ii0, ii1, ii2, ii3, ii4)
    gsems = (gs0, gs1)
    wsems = (ws0, ws1)
    ssems = (ss0, ss1)
    isems = (is0, is1, is2, is3, is4)

    pltpu.sync_copy(bins_hbm.at[pl.ds(pl.multiple_of(ebase, 8), EPW)],
                    bins_all)
    pltpu.sync_copy(zrows_hbm, agg_sh.at[pl.ds(sid * SR, SR)])
    plsc.subcore_barrier()

    def fire_idx(ck, v):
        off = pl.multiple_of(ebase + ck * C, 8)
        pltpu.async_copy(idxj_hbm.at[pl.ds(off, C)], ijs[v], isems[v])
        pltpu.async_copy(idxi_hbm.at[pl.ds(off, C)], iis[v], isems[v])

    def wait_idx(ck, v):
        off = pl.multiple_of(ebase + ck * C, 8)
        pltpu.make_async_copy(idxj_hbm.at[pl.ds(off, C)], ijs[v],
                              isems[v]).wait()
        pltpu.make_async_copy(idxi_hbm.at[pl.ds(off, C)], iis[v],
                              isems[v]).wait()

    def fire_data(ck, b, v):
        loc = pl.multiple_of(ck * C, 8)
        pltpu.async_copy(wtab_hbm.at[bins_all.at[pl.ds(loc, C)]], wbs[b],
                         wsems[b])
        pltpu.async_copy(h_hbm.at[ijs[v]], hbs[b], gsems[b])

    def drain_scatter(b, v):
        pltpu.make_async_copy(prods[b], agg_sh.at[iis[v]], ssems[b]).wait()

    def process(ck, b, v):
        loc = pl.multiple_of(ck * C, 8)
        pltpu.make_async_copy(wtab_hbm.at[bins_all.at[pl.ds(loc, C)]],
                              wbs[b], wsems[b]).wait()
        pltpu.make_async_copy(h_hbm.at[ijs[v]], hbs[b], gsems[b]).wait()

        @pl.when(ck >= NBUF)
        def _():
            drain_scatter(b, (v + 3) % 5)

        @plsc.parallel_loop(0, C, 1, unroll=2)
        def _(i):
            for m in range(D // 16):
                sl = pl.ds(16 * m, 16)
                prods[b][i, sl] = hbs[b][i, sl] * wbs[b][i, sl]

        pltpu.async_copy(prods[b], agg_sh.at[iis[v]], ssems[b], add=True)

        @pl.when(ck + 3 < NCH)
        def _():
            fire_idx(ck + 3, (v + 3) % 5)

        @pl.when(ck + NBUF < NCH)
        def _():
            wait_idx(ck + NBUF, (v + 2) % 5)
            fire_data(ck + NBUF, b, (v + 2) % 5)

    for v in range(3):
        fire_idx(v, v)
    for b in range(NBUF):
        wait_idx(b, b)
        fire_data(b, b, b)

    def outer(k, c):
        for u in range(10):
            process(k * 10 + u, u % 2, u % 5)
        return c

    lax.fori_loop(0, NCH // 10, outer, 0)
    for b in range(NBUF):
        ck = NCH - NBUF + b
        drain_scatter(b if ck % 2 == b else 1 - b, ck % 5)
    plsc.subcore_barrier()

    pltpu.sync_copy(agg_sh.at[pl.ds(sid * SR, SR)],
                    out_hbm.at[cid, pl.ds(sid * SR, SR)])


def _sc_edge(h, wtab, bins, idx_i, idx_j, zrows):
    mesh = plsc.VectorSubcoreMesh(core_axis_name="c", subcore_axis_name="s")
    kern = pl.kernel(
        _sc_edge_body,
        out_type=jax.ShapeDtypeStruct((NC, NP, D), _F32),
        mesh=mesh,
        scratch_types=(
            [pltpu.VMEM((EPW,), jnp.int32)]
            + [pltpu.VMEM((C, D), _F32)] * 6
            + [pltpu.VMEM((C,), jnp.int32)] * 10
            + [pltpu.VMEM_SHARED((NP, D), _F32)]
            + [pltpu.SemaphoreType.DMA] * 11
        ),
        compiler_params=pltpu.CompilerParams(needs_layout_passes=False),
    )
    return kern(h, wtab, bins, idx_i, idx_j, zrows)


def _mlp_body(agg_ref, x_ref, wo1_ref, bo1_ref, wo2_ref, bo2_ref, wnext_ref,
              x_out, h_out):
    a = agg_ref[0] + agg_ref[1]
    y = _ssp(jnp.dot(a, wo1_ref[...], preferred_element_type=_F32) + bo1_ref[...])
    v = jnp.dot(y, wo2_ref[...], preferred_element_type=_F32) + bo2_ref[...]
    xn = x_ref[...] + v
    x_out[...] = xn
    h_out[...] = jnp.dot(xn, wnext_ref[...], preferred_element_type=_F32)


def _mlp(agg, x, wo1, bo1, wo2, bo2, wnext):
    grid = (N // BN,)
    nb = pl.BlockSpec((BN, D), lambda i: (i, 0))
    full = pl.BlockSpec((D, D), lambda i: (0, 0))
    bias = pl.BlockSpec((1, D), lambda i: (0, 0))
    return pl.pallas_call(
        _mlp_body,
        grid=grid,
        in_specs=[
            pl.BlockSpec((NC, BN, D), lambda i: (0, i, 0)),
            nb, full, bias, full, bias, full,
        ],
        out_specs=[nb, nb],
        out_shape=[jax.ShapeDtypeStruct((N, D), _F32)] * 2,
    )(agg, x, wo1, bo1.reshape(1, D), wo2, bo2.reshape(1, D), wnext)


def kernel(Z, Rij, idx_i, idx_j, emb, W_in2f, Wf1, bf1, Wf2, bf2,
           Wo1, bo1, Wo2, bo2):
    idx_i = idx_i.astype(jnp.int32)
    idx_j = idx_j.astype(jnp.int32)
    tabs = _tables(Wf1, bf1, Wf2, bf2)
    zpad = jnp.zeros((TPAD, D), _F32)
    tabs = [jnp.concatenate([tab, zpad], axis=0) for tab in tabs]
    bins = _bins(Rij).reshape(E)
    x, h = _embed_init(Z, emb, W_in2f[0])
    zrows = jnp.zeros((SR, D), _F32)
    for t in range(T):
        agg = _sc_edge(h, tabs[t], bins, idx_i, idx_j, zrows)
        x, h = _mlp(agg, x, Wo1[t], bo1[t], Wo2[t], bo2[t],
                    W_in2f[(t + 1) % T])
    return x


# speedup vs baseline: 1.0495x; 1.0032x over previous
import functools

import jax
import jax.numpy as jnp
import numpy as np
from jax import lax
from jax.experimental import pallas as pl
from jax.experimental.pallas import tpu as pltpu
from jax.experimental.pallas import tpu_sc as plsc

N = 10000
E = 320000
D = 128
NRBF = 20
T = 3
CUTOFF = 5.0
MAX_Z = 100

NC = 2
NS = 16
NW = NC * NS
EPW = E // NW
C = 40
NCH = EPW // C
NBUF = 2
NP = 10240
SR = NP // NS

BE = 4000
BN = 1000

_F32 = jnp.float32

def _ssp(x):
    return jax.nn.softplus(x) - jnp.log(2.0).astype(_F32)


S = 4096
TPAD = 8
DELTA = CUTOFF / S


TAB = S + TPAD


def _prelude_body(rij_ref, z_ref, emb_ref, win_ref,
                  wf1_ref, bf1_ref, wf2_ref, bf2_ref,
                  bins_out, x_out, h_out, o0, o1, o2):
    i = pl.program_id(0)
    r = rij_ref[...]
    d = jnp.sqrt(jnp.sum(r * r, axis=1, keepdims=True) + 1e-12)
    bins_out[...] = jnp.minimum((d * (1.0 / DELTA)).astype(jnp.int32), S)

    @pl.when(i < N // BN)
    def _():
        z = z_ref[...]
        classes = lax.broadcasted_iota(jnp.int32, (BN, MAX_Z), 1)
        onehot = (z == classes).astype(_F32)
        x = jnp.dot(onehot, emb_ref[...], preferred_element_type=_F32)
        x_out[...] = x
        h_out[...] = jnp.dot(x, win_ref[...], preferred_element_type=_F32)

    @pl.when(i == 0)
    def _():
        k = lax.broadcasted_iota(jnp.int32, (TAB, 1), 0).astype(_F32)
        d = (k + 0.5) * DELTA
        step = CUTOFF / (NRBF - 1)
        offs = lax.broadcasted_iota(jnp.int32, (1, NRBF), 1).astype(_F32) * step
        coeff = -0.5 / step**2
        f = jnp.exp(coeff * (d - offs) ** 2)
        rcut = 0.5 * (jnp.cos(d * (jnp.pi / CUTOFF)) + 1.0)
        rcut = rcut * (d < CUTOFF).astype(_F32)
        wf1 = wf1_ref[...]
        bf1 = bf1_ref[...]
        wf2 = wf2_ref[...]
        bf2 = bf2_ref[...]
        for t, o in enumerate((o0, o1, o2)):
            y = _ssp(jnp.dot(f, wf1[t], preferred_element_type=_F32)
                     + bf1[t][None, :])
            w = jnp.dot(y, wf2[t], preferred_element_type=_F32) + bf2[t][None, :]
            o[...] = w * rcut


def _prelude(Rij, Z, emb, w_in2f0, Wf1, bf1, Wf2, bf2):
    nblk = N // BN
    nmap = lambda i: (jnp.minimum(i, nblk - 1), 0)
    tb = pl.BlockSpec((TAB, D), lambda i: (0, 0))
    return pl.pallas_call(
        _prelude_body,
        grid=(E // BE,),
        in_specs=[
            pl.BlockSpec((BE, 3), lambda i: (i, 0)),
            pl.BlockSpec((BN, 1), nmap),
            pl.BlockSpec((MAX_Z, D), lambda i: (0, 0)),
            pl.BlockSpec((D, D), lambda i: (0, 0)),
            pl.BlockSpec((T, NRBF, D), lambda i: (0, 0, 0)),
            pl.BlockSpec((T, D), lambda i: (0, 0)),
            pl.BlockSpec((T, D, D), lambda i: (0, 0, 0)),
            pl.BlockSpec((T, D), lambda i: (0, 0)),
        ],
        out_specs=[
            pl.BlockSpec((BE, 1), lambda i: (i, 0)),
            pl.BlockSpec((BN, D), nmap),
            pl.BlockSpec((BN, D), nmap),
            tb, tb, tb,
        ],
        out_shape=[
            jax.ShapeDtypeStruct((E, 1), jnp.int32),
            jax.ShapeDtypeStruct((N, D), _F32),
            jax.ShapeDtypeStruct((N, D), _F32),
        ] + [jax.ShapeDtypeStruct((TAB, D), _F32)] * T,
    )(Rij, Z.reshape(N, 1).astype(jnp.int32), emb, w_in2f0,
      Wf1, bf1, Wf2, bf2)


def _sc_edge_body(h_hbm, wtab_hbm, bins_hbm, idxi_hbm, idxj_hbm, zrows_hbm,
                  out_hbm,
                  bins_all, hb0, hb1, wb0, wb1, pr0, pr1,
                  ij0, ij1, ij2, ij3, ij4, ii0, ii1, ii2, ii3, ii4,
                  agg_sh,
                  gs0, gs1, ws0, ws1, ss0, ss1, is0, is1, is2, is3, is4):
    cid = lax.axis_index("c")
    sid = lax.axis_index("s")
    wid = cid * NS + sid
    ebase = wid * EPW
    hbs = (hb0, hb1)
    wbs = (wb0, wb1)
    prods = (pr0, pr1)
    ijs = (ij0, ij1, ij2, ij3, ij4)
    iis = (ii0, ii1, ii2, ii3, ii4)
    gsems = (gs0, gs1)
    wsems = (ws0, ws1)
    ssems = (ss0, ss1)
    isems = (is0, is1, is2, is3, is4)

    pltpu.sync_copy(bins_hbm.at[pl.ds(pl.multiple_of(ebase, 8), EPW)],
                    bins_all)
    pltpu.sync_copy(zrows_hbm, agg_sh.at[pl.ds(sid * SR, SR)])
    plsc.subcore_barrier()

    def fire_idx(ck, v):
        off = pl.multiple_of(ebase + ck * C, 8)
        pltpu.async_copy(idxj_hbm.at[pl.ds(off, C)], ijs[v], isems[v])
        pltpu.async_copy(idxi_hbm.at[pl.ds(off, C)], iis[v], isems[v])

    def wait_idx(ck, v):
        off = pl.multiple_of(ebase + ck * C, 8)
        pltpu.make_async_copy(idxj_hbm.at[pl.ds(off, C)], ijs[v],
                              isems[v]).wait()
        pltpu.make_async_copy(idxi_hbm.at[pl.ds(off, C)], iis[v],
                              isems[v]).wait()

    def fire_data(ck, b, v):
        loc = pl.multiple_of(ck * C, 8)
        pltpu.async_copy(wtab_hbm.at[bins_all.at[pl.ds(loc, C)]], wbs[b],
                         wsems[b])
        pltpu.async_copy(h_hbm.at[ijs[v]], hbs[b], gsems[b])

    def drain_scatter(b, v):
        pltpu.make_async_copy(prods[b], agg_sh.at[iis[v]], ssems[b]).wait()

    def process(ck, b, v):
        loc = pl.multiple_of(ck * C, 8)
        pltpu.make_async_copy(wtab_hbm.at[bins_all.at[pl.ds(loc, C)]],
                              wbs[b], wsems[b]).wait()
        pltpu.make_async_copy(h_hbm.at[ijs[v]], hbs[b], gsems[b]).wait()

        @pl.when(ck >= NBUF)
        def _():
            drain_scatter(b, (v + 3) % 5)

        @plsc.parallel_loop(0, C, 1, unroll=2)
        def _(i):
            for m in range(D // 16):
                sl = pl.ds(16 * m, 16)
                prods[b][i, sl] = hbs[b][i, sl] * wbs[b][i, sl]

        pltpu.async_copy(prods[b], agg_sh.at[iis[v]], ssems[b], add=True)

        @pl.when(ck + 3 < NCH)
        def _():
            fire_idx(ck + 3, (v + 3) % 5)

        @pl.when(ck + NBUF < NCH)
        def _():
            wait_idx(ck + NBUF, (v + 2) % 5)
            fire_data(ck + NBUF, b, (v + 2) % 5)

    for v in range(3):
        fire_idx(v, v)
    for b in range(NBUF):
        wait_idx(b, b)
        fire_data(b, b, b)

    def outer(k, c):
        for u in range(10):
            process(k * 10 + u, u % 2, u % 5)
        return c

    lax.fori_loop(0, NCH // 10, outer, 0)
    for b in range(NBUF):
        ck = NCH - NBUF + b
        drain_scatter(b if ck % 2 == b else 1 - b, ck % 5)
    plsc.subcore_barrier()

    pltpu.sync_copy(agg_sh.at[pl.ds(sid * SR, SR)],
                    out_hbm.at[cid, pl.ds(sid * SR, SR)])


def _sc_edge(h, wtab, bins, idx_i, idx_j, zrows):
    mesh = plsc.VectorSubcoreMesh(core_axis_name="c", subcore_axis_name="s")
    kern = pl.kernel(
        _sc_edge_body,
        out_type=jax.ShapeDtypeStruct((NC, NP, D), _F32),
        mesh=mesh,
        scratch_types=(
            [pltpu.VMEM((EPW,), jnp.int32)]
            + [pltpu.VMEM((C, D), _F32)] * 6
            + [pltpu.VMEM((C,), jnp.int32)] * 10
            + [pltpu.VMEM_SHARED((NP, D), _F32)]
            + [pltpu.SemaphoreType.DMA] * 11
        ),
        compiler_params=pltpu.CompilerParams(needs_layout_passes=False),
    )
    return kern(h, wtab, bins, idx_i, idx_j, zrows)


def _mlp_body(agg_ref, x_ref, wo1_ref, bo1_ref, wo2_ref, bo2_ref, wnext_ref,
              x_out, h_out):
    a = agg_ref[0] + agg_ref[1]
    y = _ssp(jnp.dot(a, wo1_ref[...], preferred_element_type=_F32) + bo1_ref[...])
    v = jnp.dot(y, wo2_ref[...], preferred_element_type=_F32) + bo2_ref[...]
    xn = x_ref[...] + v
    x_out[...] = xn
    h_out[...] = jnp.dot(xn, wnext_ref[...], preferred_element_type=_F32)


def _mlp(agg, x, wo1, bo1, wo2, bo2, wnext):
    grid = (N // BN,)
    nb = pl.BlockSpec((BN, D), lambda i: (i, 0))
    full = pl.BlockSpec((D, D), lambda i: (0, 0))
    bias = pl.BlockSpec((1, D), lambda i: (0, 0))
    return pl.pallas_call(
        _mlp_body,
        grid=grid,
        in_specs=[
            pl.BlockSpec((NC, BN, D), lambda i: (0, i, 0)),
            nb, full, bias, full, bias, full,
        ],
        out_specs=[nb, nb],
        out_shape=[jax.ShapeDtypeStruct((N, D), _F32)] * 2,
    )(agg, x, wo1, bo1.reshape(1, D), wo2, bo2.reshape(1, D), wnext)


def kernel(Z, Rij, idx_i, idx_j, emb, W_in2f, Wf1, bf1, Wf2, bf2,
           Wo1, bo1, Wo2, bo2):
    idx_i = idx_i.astype(jnp.int32)
    idx_j = idx_j.astype(jnp.int32)
    bins, x, h, tab0, tab1, tab2 = _prelude(
        Rij, Z, emb, W_in2f[0], Wf1, bf1, Wf2, bf2)
    tabs = (tab0, tab1, tab2)
    bins = bins.reshape(E)
    zrows = jnp.zeros((SR, D), _F32)
    for t in range(T):
        agg = _sc_edge(h, tabs[t], bins, idx_i, idx_j, zrows)
        x, h = _mlp(agg, x, Wo1[t], bo1[t], Wo2[t], bo2[t],
                    W_in2f[(t + 1) % T])
    return x
